# Initial kernel scaffold; baseline (speedup 1.0000x reference)
import jax
import jax.numpy as jnp
from jax.experimental import pallas as pl


def kernel(x, edge_index, W0, b0, Ws, Wout, bout):
    raise NotImplementedError("write your pallas kernel here")


# speedup vs baseline: 4.9210x; 4.9210x over previous
import functools
import math

import jax
import jax.numpy as jnp
from jax import lax
from jax.experimental import pallas as pl
from jax.experimental.pallas import tpu as pltpu
from jax.experimental.pallas import tpu_sc as plsc

_N = 10000
_E = 320000
_D = 128
_H = 128
_C = 40
_L = 8
_ALPHA = 0.1
_LAMBDA = 0.5

_NC, _NS = 2, 16
_NW = _NC * _NS
_EB = 128
_KJ = 80
_EPAD = _NW * _KJ * _EB
_NP = 10240
_RPT = _NP // _NS
_ZR = 32
_CH = 16

_mesh = plsc.VectorSubcoreMesh(
    core_axis_name="c", subcore_axis_name="s", num_cores=_NC, num_subcores=_NS
)


def _prop_body(hs, srcp, dstp, outp, srcv, dstv, rows0, rows1, zb, acc, sem0, sem1):
    c = lax.axis_index("c")
    s = lax.axis_index("s")
    wid = c * _NS + s

    def fill(r, carry):
        for k in range(_H // 16):
            zb[r, pl.ds(k * 16, 16)] = jnp.zeros((16,), jnp.float32)
        return carry

    lax.fori_loop(0, _ZR, fill, 0)
    base = s * _RPT
    def zero(t, carry):
        pltpu.sync_copy(zb, acc.at[pl.ds(base + t * _ZR, _ZR)])
        return carry

    lax.fori_loop(0, _RPT // _ZR, zero, 0)
    plsc.subcore_barrier()

    def chunk(q, carry):
        pltpu.sync_copy(srcp.at[wid, pl.ds(q * _CH, _CH)], srcv)
        pltpu.sync_copy(dstp.at[wid, pl.ds(q * _CH, _CH)], dstv)
        pltpu.async_copy(hs.at[srcv.at[0]], rows0, sem0)

        def body(jj, carry2):
            j0 = 2 * jj
            j1 = j0 + 1
            pltpu.make_async_copy(hs.at[srcv.at[j0]], rows0, sem0).wait()
            pltpu.async_copy(hs.at[srcv.at[j1]], rows1, sem1)
            pltpu.sync_copy(rows0, acc.at[dstv.at[j0]], add=True)
            pltpu.make_async_copy(hs.at[srcv.at[j1]], rows1, sem1).wait()

            @pl.when(jj + 1 < _CH // 2)
            def _issue_next():
                pltpu.async_copy(hs.at[srcv.at[j0 + 2]], rows0, sem0)

            pltpu.sync_copy(rows1, acc.at[dstv.at[j1]], add=True)
            return carry2

        lax.fori_loop(0, _CH // 2, body, 0)
        return carry

    lax.fori_loop(0, _KJ // _CH, chunk, 0)
    plsc.subcore_barrier()
    pltpu.sync_copy(acc.at[pl.ds(base, _RPT)], outp.at[c, pl.ds(base, _RPT)])


_prop_call = functools.partial(
    pl.kernel,
    out_type=jax.ShapeDtypeStruct((_NC, _NP, _H), jnp.float32),
    mesh=_mesh,
    scratch_types=[
        pltpu.VMEM((_CH, _EB), jnp.int32),
        pltpu.VMEM((_CH, _EB), jnp.int32),
        pltpu.VMEM((_EB, _H), jnp.float32),
        pltpu.VMEM((_EB, _H), jnp.float32),
        pltpu.VMEM((_ZR, _H), jnp.float32),
        pltpu.VMEM_SHARED((_NP, _H), jnp.float32),
        pltpu.SemaphoreType.DMA,
        pltpu.SemaphoreType.DMA,
    ],
)(_prop_body)


def _init_body(x_ref, w0_ref, b0_ref, degp_ref, h0_ref, hs_ref, dinv_ref):
    deg = degp_ref[0][:, 0:1] + degp_ref[1][:, 0:1] + 1.0
    dinv = lax.rsqrt(deg)
    h = jnp.maximum(
        jnp.dot(x_ref[...], w0_ref[...], preferred_element_type=jnp.float32)
        + b0_ref[...],
        0.0,
    )
    h0_ref[...] = h
    hs_ref[...] = h * dinv
    dinv_ref[...] = dinv


_init_call = pl.pallas_call(
    _init_body,
    out_shape=(
        jax.ShapeDtypeStruct((_NP, _H), jnp.float32),
        jax.ShapeDtypeStruct((_NP, _H), jnp.float32),
        jax.ShapeDtypeStruct((_NP, 1), jnp.float32),
    ),
)


def _layer_body(accp_ref, hs_ref, h0_ref, dinv_ref, w_ref, hsn_ref):
    dinv = dinv_ref[...]
    agg = (accp_ref[0] + accp_ref[1] + hs_ref[...]) * dinv
    support = (1.0 - _ALPHA) * agg + _ALPHA * h0_ref[...]
    h = jnp.maximum(
        jnp.dot(support, w_ref[...], preferred_element_type=jnp.float32), 0.0
    )
    hsn_ref[...] = h * dinv


_layer_call = pl.pallas_call(
    _layer_body,
    out_shape=jax.ShapeDtypeStruct((_NP, _H), jnp.float32),
)


def _last_body(accp_ref, hs_ref, h0_ref, dinv_ref, w_ref, wout_ref, bout_ref, out_ref):
    dinv = dinv_ref[...]
    agg = (accp_ref[0] + accp_ref[1] + hs_ref[...]) * dinv
    support = (1.0 - _ALPHA) * agg + _ALPHA * h0_ref[...]
    h = jnp.maximum(
        jnp.dot(support, w_ref[...], preferred_element_type=jnp.float32), 0.0
    )
    out_ref[...] = (
        jnp.dot(h, wout_ref[...], preferred_element_type=jnp.float32) + bout_ref[...]
    )


_last_call = pl.pallas_call(
    _last_body,
    out_shape=jax.ShapeDtypeStruct((_NP, _H), jnp.float32),
)


def kernel(x, edge_index, W0, b0, Ws, Wout, bout):
    f32 = jnp.float32
    src = edge_index[0].astype(jnp.int32)
    dst = edge_index[1].astype(jnp.int32)
    pad = _EPAD - _E
    srcp = jnp.concatenate([src, jnp.zeros((pad,), jnp.int32)]).reshape(_NW, _KJ, _EB)
    dstp = jnp.concatenate([dst, jnp.full((pad,), _N, jnp.int32)]).reshape(
        _NW, _KJ, _EB
    )
    xp = jnp.zeros((_NP, _D), f32).at[:_N].set(x)

    eye = jnp.eye(_H, dtype=f32)
    betas = jnp.asarray(
        [math.log(_LAMBDA / (i + 1) + 1.0) for i in range(_L)], f32
    )
    weff = betas[:, None, None] * Ws + (1.0 - betas)[:, None, None] * eye

    wout_p = jnp.zeros((_H, _H), f32).at[:, :_C].set(Wout)
    bout_p = jnp.zeros((1, _H), f32).at[0, :_C].set(bout)

    degp = _prop_call(jnp.ones((_NP, _H), f32), srcp, dstp)
    h0, hs, dinv = _init_call(xp, W0, b0.reshape(1, _H), degp)
    for i in range(_L - 1):
        accp = _prop_call(hs, srcp, dstp)
        hs = _layer_call(accp, hs, h0, dinv, weff[i])
    accp = _prop_call(hs, srcp, dstp)
    out = _last_call(accp, hs, h0, dinv, weff[_L - 1], wout_p, bout_p)
    return out[:_N, :_C]


# speedup vs baseline: 5.2354x; 1.0639x over previous
import functools
import math

import jax
import jax.numpy as jnp
from jax import lax
from jax.experimental import pallas as pl
from jax.experimental.pallas import tpu as pltpu
from jax.experimental.pallas import tpu_sc as plsc

_N = 10000
_E = 320000
_D = 128
_H = 128
_C = 40
_L = 8
_ALPHA = 0.1
_LAMBDA = 0.5

_NC, _NS = 2, 16
_NW = _NC * _NS
_EB = 64
_KJ = 160
_EPAD = _NW * _KJ * _EB
_NP = 10240
_RPT = _NP // _NS
_ZR = 16
_CH = 32
_NB = 4

_mesh = plsc.VectorSubcoreMesh(
    core_axis_name="c", subcore_axis_name="s", num_cores=_NC, num_subcores=_NS
)


def _prop_body(hs, srcp, dstp, outp, srcv, dstv, rows, zb, acc, semg, sems):
    c = lax.axis_index("c")
    s = lax.axis_index("s")
    wid = c * _NS + s

    def fill(r, carry):
        for k in range(_H // 16):
            zb[r, pl.ds(k * 16, 16)] = jnp.zeros((16,), jnp.float32)
        return carry

    lax.fori_loop(0, _ZR, fill, 0)
    base = s * _RPT

    def zero(t, carry):
        pltpu.sync_copy(zb, acc.at[pl.ds(base + t * _ZR, _ZR)])
        return carry

    lax.fori_loop(0, _RPT // _ZR, zero, 0)
    plsc.subcore_barrier()

    def chunk(q, carry):
        pltpu.sync_copy(srcp.at[wid, pl.ds(q * _CH, _CH)], srcv)
        pltpu.sync_copy(dstp.at[wid, pl.ds(q * _CH, _CH)], dstv)
        for b in range(_NB):
            pltpu.async_copy(hs.at[srcv.at[b]], rows[b], semg[b])

        def group(g, carry2):
            j = g * _NB
            for b in range(_NB):
                pltpu.make_async_copy(hs.at[srcv.at[j + b]], rows[b], semg[b]).wait()
                pltpu.async_copy(rows[b], acc.at[dstv.at[j + b]], sems[b], add=True)
            for b in range(_NB):
                pltpu.make_async_copy(rows[b], acc.at[dstv.at[j + b]], sems[b]).wait()

                @pl.when(j + _NB + b < _CH)
                def _issue_next(b=b):
                    pltpu.async_copy(hs.at[srcv.at[j + _NB + b]], rows[b], semg[b])

            return carry2

        lax.fori_loop(0, _CH // _NB, group, 0)
        return carry

    lax.fori_loop(0, _KJ // _CH, chunk, 0)
    plsc.subcore_barrier()
    pltpu.sync_copy(acc.at[pl.ds(base, _RPT)], outp.at[c, pl.ds(base, _RPT)])


_prop_call = functools.partial(
    pl.kernel,
    out_type=jax.ShapeDtypeStruct((_NC, _NP, _H), jnp.float32),
    mesh=_mesh,
    scratch_types=[
        pltpu.VMEM((_CH, _EB), jnp.int32),
        pltpu.VMEM((_CH, _EB), jnp.int32),
        [pltpu.VMEM((_EB, _H), jnp.float32) for _ in range(_NB)],
        pltpu.VMEM((_ZR, _H), jnp.float32),
        pltpu.VMEM_SHARED((_NP, _H), jnp.float32),
        [pltpu.SemaphoreType.DMA for _ in range(_NB)],
        [pltpu.SemaphoreType.DMA for _ in range(_NB)],
    ],
)(_prop_body)


def _init_body(x_ref, w0_ref, b0_ref, degp_ref, h0_ref, hs_ref, dinv_ref):
    deg = degp_ref[0][:, 0:1] + degp_ref[1][:, 0:1] + 1.0
    dinv = lax.rsqrt(deg)
    h = jnp.maximum(
        jnp.dot(x_ref[...], w0_ref[...], preferred_element_type=jnp.float32)
        + b0_ref[...],
        0.0,
    )
    h0_ref[...] = h
    hs_ref[...] = h * dinv
    dinv_ref[...] = dinv


_init_call = pl.pallas_call(
    _init_body,
    out_shape=(
        jax.ShapeDtypeStruct((_NP, _H), jnp.float32),
        jax.ShapeDtypeStruct((_NP, _H), jnp.float32),
        jax.ShapeDtypeStruct((_NP, 1), jnp.float32),
    ),
)


def _layer_body(accp_ref, hs_ref, h0_ref, dinv_ref, w_ref, hsn_ref):
    dinv = dinv_ref[...]
    agg = (accp_ref[0] + accp_ref[1] + hs_ref[...]) * dinv
    support = (1.0 - _ALPHA) * agg + _ALPHA * h0_ref[...]
    h = jnp.maximum(
        jnp.dot(support, w_ref[...], preferred_element_type=jnp.float32), 0.0
    )
    hsn_ref[...] = h * dinv


_layer_call = pl.pallas_call(
    _layer_body,
    out_shape=jax.ShapeDtypeStruct((_NP, _H), jnp.float32),
)


def _last_body(accp_ref, hs_ref, h0_ref, dinv_ref, w_ref, wout_ref, bout_ref, out_ref):
    dinv = dinv_ref[...]
    agg = (accp_ref[0] + accp_ref[1] + hs_ref[...]) * dinv
    support = (1.0 - _ALPHA) * agg + _ALPHA * h0_ref[...]
    h = jnp.maximum(
        jnp.dot(support, w_ref[...], preferred_element_type=jnp.float32), 0.0
    )
    out_ref[...] = (
        jnp.dot(h, wout_ref[...], preferred_element_type=jnp.float32) + bout_ref[...]
    )


_last_call = pl.pallas_call(
    _last_body,
    out_shape=jax.ShapeDtypeStruct((_NP, _H), jnp.float32),
)


def kernel(x, edge_index, W0, b0, Ws, Wout, bout):
    f32 = jnp.float32
    src = edge_index[0].astype(jnp.int32)
    dst = edge_index[1].astype(jnp.int32)
    pad = _EPAD - _E
    srcp = jnp.concatenate([src, jnp.zeros((pad,), jnp.int32)]).reshape(_NW, _KJ, _EB)
    dstp = jnp.concatenate([dst, jnp.full((pad,), _N, jnp.int32)]).reshape(
        _NW, _KJ, _EB
    )
    xp = jnp.zeros((_NP, _D), f32).at[:_N].set(x)

    eye = jnp.eye(_H, dtype=f32)
    betas = jnp.asarray(
        [math.log(_LAMBDA / (i + 1) + 1.0) for i in range(_L)], f32
    )
    weff = betas[:, None, None] * Ws + (1.0 - betas)[:, None, None] * eye

    wout_p = jnp.zeros((_H, _H), f32).at[:, :_C].set(Wout)
    bout_p = jnp.zeros((1, _H), f32).at[0, :_C].set(bout)

    degp = _prop_call(jnp.ones((_NP, _H), f32), srcp, dstp)
    h0, hs, dinv = _init_call(xp, W0, b0.reshape(1, _H), degp)
    for i in range(_L - 1):
        accp = _prop_call(hs, srcp, dstp)
        hs = _layer_call(accp, hs, h0, dinv, weff[i])
    accp = _prop_call(hs, srcp, dstp)
    out = _last_call(accp, hs, h0, dinv, weff[_L - 1], wout_p, bout_p)
    return out[:_N, :_C]


# speedup vs baseline: 5.7596x; 1.1001x over previous
import functools
import math

import jax
import jax.numpy as jnp
from jax import lax
from jax.experimental import pallas as pl
from jax.experimental.pallas import tpu as pltpu
from jax.experimental.pallas import tpu_sc as plsc

_N = 10000
_E = 320000
_D = 128
_H = 128
_C = 40
_L = 8
_ALPHA = 0.1
_LAMBDA = 0.5

_NC, _NS = 2, 16
_NW = _NC * _NS
_EB = 64
_KJ = 160
_EPAD = _NW * _KJ * _EB
_NP = 10240
_RPT = _NP // _NS
_ZR = 16
_CH = 32
_NB = 4

_mesh = plsc.VectorSubcoreMesh(
    core_axis_name="c", subcore_axis_name="s", num_cores=_NC, num_subcores=_NS
)


def _prop_body(hs, srcp, dstp, outp, srcv, dstv, rows, zb, acc, semg, sems):
    c = lax.axis_index("c")
    s = lax.axis_index("s")
    wid = c * _NS + s

    def fill(r, carry):
        for k in range(_H // 16):
            zb[r, pl.ds(k * 16, 16)] = jnp.zeros((16,), jnp.float32)
        return carry

    lax.fori_loop(0, _ZR, fill, 0)
    base = s * _RPT

    def zero(t, carry):
        pltpu.sync_copy(zb, acc.at[pl.ds(base + t * _ZR, _ZR)])
        return carry

    lax.fori_loop(0, _RPT // _ZR, zero, 0)
    plsc.subcore_barrier()

    def chunk(q, carry):
        pltpu.sync_copy(srcp.at[wid, pl.ds(q * _CH, _CH)], srcv)
        pltpu.sync_copy(dstp.at[wid, pl.ds(q * _CH, _CH)], dstv)
        for b in range(_NB):
            pltpu.async_copy(hs.at[srcv.at[b]], rows[b], semg[b])

        def group(g, carry2):
            j = g * _NB
            for b in range(_NB):
                pltpu.make_async_copy(hs.at[srcv.at[j + b]], rows[b], semg[b]).wait()
                pltpu.async_copy(rows[b], acc.at[dstv.at[j + b]], sems[b], add=True)
            for b in range(_NB):
                pltpu.make_async_copy(rows[b], acc.at[dstv.at[j + b]], sems[b]).wait()

                @pl.when(j + _NB + b < _CH)
                def _issue_next(b=b):
                    pltpu.async_copy(hs.at[srcv.at[j + _NB + b]], rows[b], semg[b])

            return carry2

        lax.fori_loop(0, _CH // _NB, group, 0)
        return carry

    lax.fori_loop(0, _KJ // _CH, chunk, 0)
    plsc.subcore_barrier()
    pltpu.sync_copy(acc.at[pl.ds(base, _RPT)], outp.at[c, pl.ds(base, _RPT)])


_prop_call = functools.partial(
    pl.kernel,
    out_type=jax.ShapeDtypeStruct((_NC, _NP, _H), jnp.float32),
    mesh=_mesh,
    scratch_types=[
        pltpu.VMEM((_CH, _EB), jnp.int32),
        pltpu.VMEM((_CH, _EB), jnp.int32),
        [pltpu.VMEM((_EB, _H), jnp.float32) for _ in range(_NB)],
        pltpu.VMEM((_ZR, _H), jnp.float32),
        pltpu.VMEM_SHARED((_NP, _H), jnp.float32),
        [pltpu.SemaphoreType.DMA for _ in range(_NB)],
        [pltpu.SemaphoreType.DMA for _ in range(_NB)],
    ],
)(_prop_body)


def _deg_body(dstp, outp, dstv, ones, zb, acc, sems):
    c = lax.axis_index("c")
    s = lax.axis_index("s")
    wid = c * _NS + s

    def fill(r, carry):
        for k in range(_H // 16):
            zb[r, pl.ds(k * 16, 16)] = jnp.zeros((16,), jnp.float32)
        return carry

    lax.fori_loop(0, _ZR, fill, 0)

    def fill1(r, carry):
        for k in range(_H // 16):
            ones[r, pl.ds(k * 16, 16)] = jnp.ones((16,), jnp.float32)
        return carry

    lax.fori_loop(0, _EB, fill1, 0)
    base = s * _RPT

    def zero(t, carry):
        pltpu.sync_copy(zb, acc.at[pl.ds(base + t * _ZR, _ZR)])
        return carry

    lax.fori_loop(0, _RPT // _ZR, zero, 0)
    plsc.subcore_barrier()

    def chunk(q, carry):
        pltpu.sync_copy(dstp.at[wid, pl.ds(q * _CH, _CH)], dstv)

        def group(g, carry2):
            j = g * _NB
            for b in range(_NB):
                pltpu.async_copy(ones, acc.at[dstv.at[j + b]], sems[b], add=True)
            for b in range(_NB):
                pltpu.make_async_copy(ones, acc.at[dstv.at[j + b]], sems[b]).wait()
            return carry2

        lax.fori_loop(0, _CH // _NB, group, 0)
        return carry

    lax.fori_loop(0, _KJ // _CH, chunk, 0)
    plsc.subcore_barrier()
    pltpu.sync_copy(acc.at[pl.ds(base, _RPT)], outp.at[c, pl.ds(base, _RPT)])


_deg_call = functools.partial(
    pl.kernel,
    out_type=jax.ShapeDtypeStruct((_NC, _NP, _H), jnp.float32),
    mesh=_mesh,
    scratch_types=[
        pltpu.VMEM((_CH, _EB), jnp.int32),
        pltpu.VMEM((_EB, _H), jnp.float32),
        pltpu.VMEM((_ZR, _H), jnp.float32),
        pltpu.VMEM_SHARED((_NP, _H), jnp.float32),
        [pltpu.SemaphoreType.DMA for _ in range(_NB)],
    ],
)(_deg_body)


def _init_body(x_ref, w0_ref, b0_ref, degp_ref, h0_ref, hs_ref, dinv_ref):
    deg = degp_ref[0][:, 0:1] + degp_ref[1][:, 0:1] + 1.0
    dinv = lax.rsqrt(deg)
    h = jnp.maximum(
        jnp.dot(x_ref[...], w0_ref[...], preferred_element_type=jnp.float32)
        + b0_ref[...],
        0.0,
    )
    h0_ref[...] = h
    hs_ref[...] = h * dinv
    dinv_ref[...] = dinv


_init_call = pl.pallas_call(
    _init_body,
    out_shape=(
        jax.ShapeDtypeStruct((_NP, _H), jnp.float32),
        jax.ShapeDtypeStruct((_NP, _H), jnp.float32),
        jax.ShapeDtypeStruct((_NP, 1), jnp.float32),
    ),
)


def _layer_body(accp_ref, hs_ref, h0_ref, dinv_ref, w_ref, hsn_ref):
    dinv = dinv_ref[...]
    agg = (accp_ref[0] + accp_ref[1] + hs_ref[...]) * dinv
    support = (1.0 - _ALPHA) * agg + _ALPHA * h0_ref[...]
    h = jnp.maximum(
        jnp.dot(support, w_ref[...], preferred_element_type=jnp.float32), 0.0
    )
    hsn_ref[...] = h * dinv


_layer_call = pl.pallas_call(
    _layer_body,
    out_shape=jax.ShapeDtypeStruct((_NP, _H), jnp.float32),
)


def _last_body(accp_ref, hs_ref, h0_ref, dinv_ref, w_ref, wout_ref, bout_ref, out_ref):
    dinv = dinv_ref[...]
    agg = (accp_ref[0] + accp_ref[1] + hs_ref[...]) * dinv
    support = (1.0 - _ALPHA) * agg + _ALPHA * h0_ref[...]
    h = jnp.maximum(
        jnp.dot(support, w_ref[...], preferred_element_type=jnp.float32), 0.0
    )
    out_ref[...] = (
        jnp.dot(h, wout_ref[...], preferred_element_type=jnp.float32) + bout_ref[...]
    )


_last_call = pl.pallas_call(
    _last_body,
    out_shape=jax.ShapeDtypeStruct((_NP, _H), jnp.float32),
)


def kernel(x, edge_index, W0, b0, Ws, Wout, bout):
    f32 = jnp.float32
    src = edge_index[0].astype(jnp.int32)
    dst = edge_index[1].astype(jnp.int32)
    pad = _EPAD - _E
    srcp = jnp.concatenate([src, jnp.zeros((pad,), jnp.int32)]).reshape(_NW, _KJ, _EB)
    dstp = jnp.concatenate([dst, jnp.full((pad,), _N, jnp.int32)]).reshape(
        _NW, _KJ, _EB
    )
    xp = jnp.zeros((_NP, _D), f32).at[:_N].set(x)

    eye = jnp.eye(_H, dtype=f32)
    betas = jnp.asarray(
        [math.log(_LAMBDA / (i + 1) + 1.0) for i in range(_L)], f32
    )
    weff = betas[:, None, None] * Ws + (1.0 - betas)[:, None, None] * eye

    wout_p = jnp.zeros((_H, _H), f32).at[:, :_C].set(Wout)
    bout_p = jnp.zeros((1, _H), f32).at[0, :_C].set(bout)

    degp = _deg_call(dstp)
    h0, hs, dinv = _init_call(xp, W0, b0.reshape(1, _H), degp)
    for i in range(_L - 1):
        accp = _prop_call(hs, srcp, dstp)
        hs = _layer_call(accp, hs, h0, dinv, weff[i])
    accp = _prop_call(hs, srcp, dstp)
    out = _last_call(accp, hs, h0, dinv, weff[_L - 1], wout_p, bout_p)
    return out[:_N, :_C]


# speedup vs baseline: 5.8543x; 1.0164x over previous
import functools
import math

import jax
import jax.numpy as jnp
from jax import lax
from jax.experimental import pallas as pl
from jax.experimental.pallas import tpu as pltpu
from jax.experimental.pallas import tpu_sc as plsc

_N = 10000
_E = 320000
_D = 128
_H = 128
_C = 40
_L = 8
_ALPHA = 0.1
_LAMBDA = 0.5

_NC, _NS = 2, 16
_NW = _NC * _NS
_EB = 32
_KJ = 320
_EPAD = _NW * _KJ * _EB
_NP = 10240
_RPT = _NP // _NS
_ZR = 16
_CH = 32
_NB = 4

_mesh = plsc.VectorSubcoreMesh(
    core_axis_name="c", subcore_axis_name="s", num_cores=_NC, num_subcores=_NS
)


def _prop_body(hs, srcp, dstp, outp, srcv, dstv, rows, zb, acc, semg, sems):
    c = lax.axis_index("c")
    s = lax.axis_index("s")
    wid = c * _NS + s

    def fill(r, carry):
        for k in range(_H // 16):
            zb[r, pl.ds(k * 16, 16)] = jnp.zeros((16,), jnp.float32)
        return carry

    lax.fori_loop(0, _ZR, fill, 0)
    base = s * _RPT

    def zero(t, carry):
        pltpu.sync_copy(zb, acc.at[pl.ds(base + t * _ZR, _ZR)])
        return carry

    lax.fori_loop(0, _RPT // _ZR, zero, 0)
    plsc.subcore_barrier()

    def chunk(q, carry):
        pltpu.sync_copy(srcp.at[wid, pl.ds(q * _CH, _CH)], srcv)
        pltpu.sync_copy(dstp.at[wid, pl.ds(q * _CH, _CH)], dstv)
        for b in range(_NB):
            pltpu.async_copy(hs.at[srcv.at[b]], rows[b], semg[b])

        def group(g, carry2):
            j = g * _NB
            for b in range(_NB):
                pltpu.make_async_copy(hs.at[srcv.at[j + b]], rows[b], semg[b]).wait()
                pltpu.async_copy(rows[b], acc.at[dstv.at[j + b]], sems[b], add=True)
            for b in range(_NB):
                pltpu.make_async_copy(rows[b], acc.at[dstv.at[j + b]], sems[b]).wait()

                @pl.when(j + _NB + b < _CH)
                def _issue_next(b=b):
                    pltpu.async_copy(hs.at[srcv.at[j + _NB + b]], rows[b], semg[b])

            return carry2

        lax.fori_loop(0, _CH // _NB, group, 0)
        return carry

    lax.fori_loop(0, _KJ // _CH, chunk, 0)
    plsc.subcore_barrier()
    pltpu.sync_copy(acc.at[pl.ds(base, _RPT)], outp.at[c, pl.ds(base, _RPT)])


_prop_call = functools.partial(
    pl.kernel,
    out_type=jax.ShapeDtypeStruct((_NC, _NP, _H), jnp.float32),
    mesh=_mesh,
    scratch_types=[
        pltpu.VMEM((_CH, _EB), jnp.int32),
        pltpu.VMEM((_CH, _EB), jnp.int32),
        [pltpu.VMEM((_EB, _H), jnp.float32) for _ in range(_NB)],
        pltpu.VMEM((_ZR, _H), jnp.float32),
        pltpu.VMEM_SHARED((_NP, _H), jnp.float32),
        [pltpu.SemaphoreType.DMA for _ in range(_NB)],
        [pltpu.SemaphoreType.DMA for _ in range(_NB)],
    ],
)(_prop_body)


def _deg_body(dstp, outp, dstv, ones, zb, acc, sems):
    c = lax.axis_index("c")
    s = lax.axis_index("s")
    wid = c * _NS + s

    def fill(r, carry):
        for k in range(_H // 16):
            zb[r, pl.ds(k * 16, 16)] = jnp.zeros((16,), jnp.float32)
        return carry

    lax.fori_loop(0, _ZR, fill, 0)

    def fill1(r, carry):
        for k in range(_H // 16):
            ones[r, pl.ds(k * 16, 16)] = jnp.ones((16,), jnp.float32)
        return carry

    lax.fori_loop(0, _EB, fill1, 0)
    base = s * _RPT

    def zero(t, carry):
        pltpu.sync_copy(zb, acc.at[pl.ds(base + t * _ZR, _ZR)])
        return carry

    lax.fori_loop(0, _RPT // _ZR, zero, 0)
    plsc.subcore_barrier()

    def chunk(q, carry):
        pltpu.sync_copy(dstp.at[wid, pl.ds(q * _CH, _CH)], dstv)

        def group(g, carry2):
            j = g * _NB
            for b in range(_NB):
                pltpu.async_copy(ones, acc.at[dstv.at[j + b]], sems[b], add=True)
            for b in range(_NB):
                pltpu.make_async_copy(ones, acc.at[dstv.at[j + b]], sems[b]).wait()
            return carry2

        lax.fori_loop(0, _CH // _NB, group, 0)
        return carry

    lax.fori_loop(0, _KJ // _CH, chunk, 0)
    plsc.subcore_barrier()
    pltpu.sync_copy(acc.at[pl.ds(base, _RPT)], outp.at[c, pl.ds(base, _RPT)])


_deg_call = functools.partial(
    pl.kernel,
    out_type=jax.ShapeDtypeStruct((_NC, _NP, _H), jnp.float32),
    mesh=_mesh,
    scratch_types=[
        pltpu.VMEM((_CH, _EB), jnp.int32),
        pltpu.VMEM((_EB, _H), jnp.float32),
        pltpu.VMEM((_ZR, _H), jnp.float32),
        pltpu.VMEM_SHARED((_NP, _H), jnp.float32),
        [pltpu.SemaphoreType.DMA for _ in range(_NB)],
    ],
)(_deg_body)


def _init_body(x_ref, w0_ref, b0_ref, degp_ref, h0_ref, hs_ref, dinv_ref):
    deg = degp_ref[0][:, 0:1] + degp_ref[1][:, 0:1] + 1.0
    dinv = lax.rsqrt(deg)
    h = jnp.maximum(
        jnp.dot(x_ref[...], w0_ref[...], preferred_element_type=jnp.float32)
        + b0_ref[...],
        0.0,
    )
    h0_ref[...] = h
    hs_ref[...] = h * dinv
    dinv_ref[...] = dinv


_init_call = pl.pallas_call(
    _init_body,
    out_shape=(
        jax.ShapeDtypeStruct((_NP, _H), jnp.float32),
        jax.ShapeDtypeStruct((_NP, _H), jnp.float32),
        jax.ShapeDtypeStruct((_NP, 1), jnp.float32),
    ),
)


def _layer_body(accp_ref, hs_ref, h0_ref, dinv_ref, w_ref, hsn_ref):
    dinv = dinv_ref[...]
    agg = (accp_ref[0] + accp_ref[1] + hs_ref[...]) * dinv
    support = (1.0 - _ALPHA) * agg + _ALPHA * h0_ref[...]
    h = jnp.maximum(
        jnp.dot(support, w_ref[...], preferred_element_type=jnp.float32), 0.0
    )
    hsn_ref[...] = h * dinv


_layer_call = pl.pallas_call(
    _layer_body,
    out_shape=jax.ShapeDtypeStruct((_NP, _H), jnp.float32),
)


def _last_body(accp_ref, hs_ref, h0_ref, dinv_ref, w_ref, wout_ref, bout_ref, out_ref):
    dinv = dinv_ref[...]
    agg = (accp_ref[0] + accp_ref[1] + hs_ref[...]) * dinv
    support = (1.0 - _ALPHA) * agg + _ALPHA * h0_ref[...]
    h = jnp.maximum(
        jnp.dot(support, w_ref[...], preferred_element_type=jnp.float32), 0.0
    )
    out_ref[...] = (
        jnp.dot(h, wout_ref[...], preferred_element_type=jnp.float32) + bout_ref[...]
    )


_last_call = pl.pallas_call(
    _last_body,
    out_shape=jax.ShapeDtypeStruct((_NP, _H), jnp.float32),
)


def kernel(x, edge_index, W0, b0, Ws, Wout, bout):
    f32 = jnp.float32
    src = edge_index[0].astype(jnp.int32)
    dst = edge_index[1].astype(jnp.int32)
    pad = _EPAD - _E
    srcp = jnp.concatenate([src, jnp.zeros((pad,), jnp.int32)]).reshape(_NW, _KJ, _EB)
    dstp = jnp.concatenate([dst, jnp.full((pad,), _N, jnp.int32)]).reshape(
        _NW, _KJ, _EB
    )
    xp = jnp.zeros((_NP, _D), f32).at[:_N].set(x)

    eye = jnp.eye(_H, dtype=f32)
    betas = jnp.asarray(
        [math.log(_LAMBDA / (i + 1) + 1.0) for i in range(_L)], f32
    )
    weff = betas[:, None, None] * Ws + (1.0 - betas)[:, None, None] * eye

    wout_p = jnp.zeros((_H, _H), f32).at[:, :_C].set(Wout)
    bout_p = jnp.zeros((1, _H), f32).at[0, :_C].set(bout)

    degp = _deg_call(dstp)
    h0, hs, dinv = _init_call(xp, W0, b0.reshape(1, _H), degp)
    for i in range(_L - 1):
        accp = _prop_call(hs, srcp, dstp)
        hs = _layer_call(accp, hs, h0, dinv, weff[i])
    accp = _prop_call(hs, srcp, dstp)
    out = _last_call(accp, hs, h0, dinv, weff[_L - 1], wout_p, bout_p)
    return out[:_N, :_C]
